# Initial kernel scaffold; baseline (speedup 1.0000x reference)
import jax
import jax.numpy as jnp
from jax.experimental import pallas as pl


def kernel(ids, att_ids, embd, W, b):
    raise NotImplementedError("write your pallas kernel here")


# speedup vs baseline: 20.4443x; 20.4443x over previous
import functools

import jax
import jax.numpy as jnp
from jax import lax
from jax.experimental import pallas as pl
from jax.experimental.pallas import tpu as pltpu
from jax.experimental.pallas import tpu_sc as plsc

VOCAB = 100000
DIM = 128
BATCH = 4096
SEQ = 200

NC = 2
NS = 16
NW = NC * NS
ROWS_PER_W = BATCH // NW
GROUPS = ROWS_PER_W // 16

VB = 1024
TC_GRID = (VOCAB + VB - 1) // VB


def _pk_body(w_ref, e_ref, p_ref):
    p_ref[...] = lax.dot_general(
        w_ref[...], e_ref[...],
        dimension_numbers=(((1,), (1,)), ((), ())),
        preferred_element_type=jnp.float32,
    )


_mesh = plsc.VectorSubcoreMesh(core_axis_name="c", subcore_axis_name="s")


@functools.partial(
    pl.kernel,
    mesh=_mesh,
    compiler_params=pltpu.CompilerParams(needs_layout_passes=False),
    out_type=jax.ShapeDtypeStruct((BATCH,), jnp.float32),
    scratch_types=[
        pltpu.VMEM((VOCAB,), jnp.float32),
        pltpu.VMEM((16 * SEQ,), jnp.int32),
        pltpu.VMEM((16 * SEQ,), jnp.float32),
        pltpu.VMEM((16,), jnp.float32),
        pltpu.VMEM((ROWS_PER_W,), jnp.float32),
    ],
)
def _sc_pool(p_hbm, ids_hbm, att_hbm, b_hbm, out_hbm,
             p_v, ids_v, att_v, b_v, out_v):
    wid = lax.axis_index("s") * NC + lax.axis_index("c")
    base = wid * ROWS_PER_W
    pltpu.sync_copy(p_hbm.at[pl.ds(0, VOCAB)], p_v)
    pltpu.sync_copy(b_hbm, b_v)
    bvec = b_v[...]
    lane_off = lax.iota(jnp.int32, 16) * SEQ
    for g in range(GROUPS):
        rb = base + g * 16
        pltpu.sync_copy(ids_hbm.at[pl.ds(rb * SEQ, 16 * SEQ)], ids_v)
        pltpu.sync_copy(att_hbm.at[pl.ds(rb * SEQ, 16 * SEQ)], att_v)

        def body(s, carry, _lane_off=lane_off):
            acc, asum = carry
            off = _lane_off + s
            idv = plsc.load_gather(ids_v, [off])
            av = plsc.load_gather(att_v, [off])
            pv = plsc.load_gather(p_v, [idv])
            return acc + av * pv, asum + av

        acc, asum = lax.fori_loop(
            0, SEQ, body,
            (jnp.zeros((16,), jnp.float32), jnp.zeros((16,), jnp.float32)))
        x = acc / asum + bvec
        out_v[pl.ds(g * 16, 16)] = 1.0 / (1.0 + jnp.exp(-x))
    pltpu.sync_copy(out_v, out_hbm.at[pl.ds(base, ROWS_PER_W)])


def kernel(ids, att_ids, embd, W, b):
    p2 = pl.pallas_call(
        _pk_body,
        grid=(TC_GRID,),
        in_specs=[
            pl.BlockSpec((1, DIM), lambda i: (0, 0)),
            pl.BlockSpec((VB, DIM), lambda i: (i, 0)),
        ],
        out_specs=pl.BlockSpec((1, VB), lambda i: (0, i)),
        out_shape=jax.ShapeDtypeStruct((1, VOCAB), jnp.float32),
    )(W, embd)
    p = p2.reshape(VOCAB)
    b16 = jnp.broadcast_to(b.astype(jnp.float32), (16,))
    return _sc_pool(p, ids.reshape(BATCH * SEQ), att_ids.reshape(BATCH * SEQ), b16)


# speedup vs baseline: 21.4075x; 1.0471x over previous
import functools

import jax
import jax.numpy as jnp
from jax import lax
from jax.experimental import pallas as pl
from jax.experimental.pallas import tpu as pltpu
from jax.experimental.pallas import tpu_sc as plsc

VOCAB = 100000
DIM = 128
BATCH = 4096
SEQ = 200

NC = 2
NS = 16
NW = NC * NS
ROWS_PER_W = BATCH // NW
GROUPS = ROWS_PER_W // 16

VB = 2048
TC_GRID = (VOCAB + VB - 1) // VB


def _pk_body(w_ref, e_ref, p_ref):
    p_ref[...] = jnp.sum(e_ref[...] * w_ref[...], axis=1)


_mesh = plsc.VectorSubcoreMesh(core_axis_name="c", subcore_axis_name="s")


@functools.partial(
    pl.kernel,
    mesh=_mesh,
    compiler_params=pltpu.CompilerParams(needs_layout_passes=False),
    out_type=jax.ShapeDtypeStruct((BATCH,), jnp.float32),
    scratch_types=[
        pltpu.VMEM((VOCAB,), jnp.float32),
        pltpu.VMEM((16, SEQ), jnp.int32),
        pltpu.VMEM((16, SEQ), jnp.float32),
        pltpu.VMEM((16,), jnp.float32),
        pltpu.VMEM((ROWS_PER_W,), jnp.float32),
    ],
)
def _sc_pool(p_hbm, ids_hbm, att_hbm, b_hbm, out_hbm,
             p_v, ids_v, att_v, b_v, out_v):
    wid = lax.axis_index("s") * NC + lax.axis_index("c")
    base = wid * ROWS_PER_W
    pltpu.sync_copy(p_hbm, p_v)
    pltpu.sync_copy(b_hbm, b_v)
    bvec = b_v[...]
    lane = lax.iota(jnp.int32, 16)
    for g in range(GROUPS):
        rb = base + g * 16
        pltpu.sync_copy(ids_hbm.at[pl.ds(rb, 16), :], ids_v)
        pltpu.sync_copy(att_hbm.at[pl.ds(rb, 16), :], att_v)

        def body(s, carry, _lane=lane):
            acc, asum = carry
            scol = jnp.full((16,), s, dtype=jnp.int32)
            idv = plsc.load_gather(ids_v, [_lane, scol])
            av = plsc.load_gather(att_v, [_lane, scol])
            pv = plsc.load_gather(p_v, [idv])
            return acc + av * pv, asum + av

        acc, asum = lax.fori_loop(
            0, SEQ, body,
            (jnp.zeros((16,), jnp.float32), jnp.zeros((16,), jnp.float32)))
        x = acc / asum + bvec
        out_v[pl.ds(g * 16, 16)] = 1.0 / (1.0 + jnp.exp(-x))
    pltpu.sync_copy(out_v, out_hbm.at[pl.ds(base, ROWS_PER_W)])


def kernel(ids, att_ids, embd, W, b):
    p = pl.pallas_call(
        _pk_body,
        grid=(TC_GRID,),
        in_specs=[
            pl.BlockSpec((1, DIM), lambda i: (0, 0)),
            pl.BlockSpec((VB, DIM), lambda i: (i, 0)),
        ],
        out_specs=pl.BlockSpec((VB,), lambda i: (i,)),
        out_shape=jax.ShapeDtypeStruct((VOCAB,), jnp.float32),
    )(W, embd)
    b16 = jnp.broadcast_to(b.astype(jnp.float32), (16,))
    return _sc_pool(p, ids, att_ids, b16)


# speedup vs baseline: 24.4418x; 1.1417x over previous
import functools

import jax
import jax.numpy as jnp
from jax import lax
from jax.experimental import pallas as pl
from jax.experimental.pallas import tpu as pltpu
from jax.experimental.pallas import tpu_sc as plsc

VOCAB = 100000
DIM = 128
BATCH = 4096
SEQ = 200

NC = 2
NS = 16
NW = NC * NS
ROWS_PER_W = BATCH // NW
GROUPS = ROWS_PER_W // 16

VB = 2048
TC_GRID = (VOCAB + VB - 1) // VB


def _pk_body(w_ref, e_ref, p_ref):
    p_ref[...] = jnp.sum(e_ref[...] * w_ref[...], axis=1)


_mesh = plsc.VectorSubcoreMesh(core_axis_name="c", subcore_axis_name="s")


@functools.partial(
    pl.kernel,
    mesh=_mesh,
    compiler_params=pltpu.CompilerParams(needs_layout_passes=False),
    out_type=jax.ShapeDtypeStruct((BATCH,), jnp.float32),
    scratch_types=[
        pltpu.VMEM((VOCAB,), jnp.float32),
        pltpu.VMEM((16, SEQ), jnp.int32),
        pltpu.VMEM((16, SEQ), jnp.int32),
        pltpu.VMEM((16, SEQ), jnp.float32),
        pltpu.VMEM((16, SEQ), jnp.float32),
        pltpu.VMEM((16,), jnp.float32),
        pltpu.VMEM((ROWS_PER_W,), jnp.float32),
        pltpu.SemaphoreType.DMA,
        pltpu.SemaphoreType.DMA,
        pltpu.SemaphoreType.DMA,
        pltpu.SemaphoreType.DMA,
    ],
)
def _sc_pool(p_hbm, ids_hbm, att_hbm, b_hbm, out_hbm,
             p_v, ids_v0, ids_v1, att_v0, att_v1, b_v, out_v,
             sem_p, sem_b, sem_g0, sem_g1):
    wid = lax.axis_index("s") * NC + lax.axis_index("c")
    base = wid * ROWS_PER_W
    bufs = [(ids_v0, att_v0, sem_g0), (ids_v1, att_v1, sem_g1)]

    def issue(g):
        iv, av, sg = bufs[g % 2]
        rb = base + g * 16
        ci = pltpu.async_copy(ids_hbm.at[pl.ds(rb, 16), :], iv, sg)
        ca = pltpu.async_copy(att_hbm.at[pl.ds(rb, 16), :], av, sg)
        return ci, ca

    cp_p = pltpu.async_copy(p_hbm, p_v, sem_p)
    cp_b = pltpu.async_copy(b_hbm, b_v, sem_b)
    pend = issue(0)
    with jax.named_scope("p_wait"):
        cp_p.wait()
        cp_b.wait()
    bvec = b_v[...]
    lane = lax.iota(jnp.int32, 16)
    zero = jnp.zeros((16,), jnp.float32)
    for g in range(GROUPS):
        iv_ref, av_ref, _ = bufs[g % 2]
        with jax.named_scope("g_wait"):
            pend[0].wait()
            pend[1].wait()
        if g + 1 < GROUPS:
            pend = issue(g + 1)

        def body(s, carry, _lane=lane, _iv=iv_ref, _av=av_ref):
            acc, asum = carry
            scol = jnp.full((16,), s, dtype=jnp.int32)
            idv = plsc.load_gather(_iv, [_lane, scol])
            av = plsc.load_gather(_av, [_lane, scol])
            pv = plsc.load_gather(p_v, [idv])
            return acc + av * pv, asum + av

        with jax.named_scope("pool"):
            acc, asum = plsc.parallel_loop(0, SEQ, unroll=4, carry=(zero, zero))(body)
            x = acc / asum + bvec
            out_v[pl.ds(g * 16, 16)] = 1.0 / (1.0 + jnp.exp(-x))
    pltpu.sync_copy(out_v, out_hbm.at[pl.ds(base, ROWS_PER_W)])


def kernel(ids, att_ids, embd, W, b):
    p = pl.pallas_call(
        _pk_body,
        grid=(TC_GRID,),
        in_specs=[
            pl.BlockSpec((1, DIM), lambda i: (0, 0)),
            pl.BlockSpec((VB, DIM), lambda i: (i, 0)),
        ],
        out_specs=pl.BlockSpec((VB,), lambda i: (i,)),
        out_shape=jax.ShapeDtypeStruct((VOCAB,), jnp.float32),
    )(W, embd)
    b16 = jnp.broadcast_to(b.astype(jnp.float32), (16,))
    return _sc_pool(p, ids, att_ids, b16)


# speedup vs baseline: 28.2281x; 1.1549x over previous
import functools

import jax
import jax.numpy as jnp
from jax import lax
from jax.experimental import pallas as pl
from jax.experimental.pallas import tpu as pltpu
from jax.experimental.pallas import tpu_sc as plsc

VOCAB = 100000
DIM = 128
BATCH = 4096
SEQ = 200

NC = 2
NS = 16
NW = NC * NS
ROWS_PER_W = BATCH // NW
GROUPS = ROWS_PER_W // 16

VB = 2048
TC_GRID = (VOCAB + VB - 1) // VB


def _pk_body(w_ref, e_ref, p_ref):
    p_ref[...] = jnp.sum(e_ref[...] * w_ref[...], axis=1)


_mesh = plsc.VectorSubcoreMesh(core_axis_name="c", subcore_axis_name="s")


@functools.partial(
    pl.kernel,
    mesh=_mesh,
    compiler_params=pltpu.CompilerParams(needs_layout_passes=False),
    out_type=jax.ShapeDtypeStruct((BATCH,), jnp.float32),
    scratch_types=[
        pltpu.VMEM((VOCAB,), jnp.float32),
        pltpu.VMEM((8, 128), jnp.int32),
        pltpu.VMEM((8, 128), jnp.int32),
        pltpu.VMEM((8, 128), jnp.float32),
        pltpu.VMEM((8, 128), jnp.float32),
        pltpu.VMEM((16,), jnp.float32),
        pltpu.VMEM((ROWS_PER_W,), jnp.float32),
        pltpu.SemaphoreType.DMA,
        pltpu.SemaphoreType.DMA,
        pltpu.SemaphoreType.DMA,
        pltpu.SemaphoreType.DMA,
    ],
)
def _sc_pool(p_hbm, idsT_hbm, attT_hbm, b_hbm, out_hbm,
             p_v, ids_v0, ids_v1, att_v0, att_v1, b_v, out_v,
             sem_p, sem_b, sem_g0, sem_g1):
    wid = lax.axis_index("s") * NC + lax.axis_index("c")
    base = wid * ROWS_PER_W
    bufs = [(ids_v0, att_v0, sem_g0), (ids_v1, att_v1, sem_g1)]
    n_st = SEQ // 8

    def issue(st):
        iv, av, sg = bufs[st % 2]
        ci = pltpu.async_copy(
            idsT_hbm.at[pl.ds(st * 8, 8), pl.ds(base, ROWS_PER_W)], iv, sg)
        ca = pltpu.async_copy(
            attT_hbm.at[pl.ds(st * 8, 8), pl.ds(base, ROWS_PER_W)], av, sg)
        return ci, ca

    cp_p = pltpu.async_copy(p_hbm, p_v, sem_p)
    cp_b = pltpu.async_copy(b_hbm, b_v, sem_b)
    pend = issue(0)
    with jax.named_scope("p_wait"):
        cp_p.wait()
        cp_b.wait()
    bvec = b_v[...]
    zero = jnp.zeros((16,), jnp.float32)
    accs = [zero] * 8
    asums = [zero] * 8
    for st in range(n_st):
        iv_ref, av_ref, _ = bufs[st % 2]
        with jax.named_scope("g_wait"):
            pend[0].wait()
            pend[1].wait()
        if st + 1 < n_st:
            pend = issue(st + 1)

        def body(s8, carry, _iv=iv_ref, _av=av_ref):
            acc, asum = list(carry[:8]), list(carry[8:])
            for lg in range(8):
                idv = _iv[s8, pl.ds(lg * 16, 16)]
                av = _av[s8, pl.ds(lg * 16, 16)]
                pv = plsc.load_gather(p_v, [idv])
                acc[lg] = acc[lg] + av * pv
                asum[lg] = asum[lg] + av
            return tuple(acc) + tuple(asum)

        with jax.named_scope("pool"):
            carry = lax.fori_loop(0, 8, body, tuple(accs) + tuple(asums))
            accs, asums = list(carry[:8]), list(carry[8:])
    for lg in range(8):
        x = accs[lg] / asums[lg] + bvec
        out_v[pl.ds(lg * 16, 16)] = 1.0 / (1.0 + jnp.exp(-x))
    pltpu.sync_copy(out_v, out_hbm.at[pl.ds(base, ROWS_PER_W)])


def kernel(ids, att_ids, embd, W, b):
    p = pl.pallas_call(
        _pk_body,
        grid=(TC_GRID,),
        in_specs=[
            pl.BlockSpec((1, DIM), lambda i: (0, 0)),
            pl.BlockSpec((VB, DIM), lambda i: (i, 0)),
        ],
        out_specs=pl.BlockSpec((VB,), lambda i: (i,)),
        out_shape=jax.ShapeDtypeStruct((VOCAB,), jnp.float32),
    )(W, embd)
    b16 = jnp.broadcast_to(b.astype(jnp.float32), (16,))
    return _sc_pool(p, ids.T, att_ids.T, b16)


# speedup vs baseline: 32.9874x; 1.1686x over previous
import functools

import jax
import jax.numpy as jnp
from jax import lax
from jax.experimental import pallas as pl
from jax.experimental.pallas import tpu as pltpu
from jax.experimental.pallas import tpu_sc as plsc

VOCAB = 100000
DIM = 128
BATCH = 4096
SEQ = 200

NC = 2
NS = 16
NW = NC * NS
ROWS_PER_W = BATCH // NW
GROUPS = ROWS_PER_W // 16

VB = 2048
TC_GRID = (VOCAB + VB - 1) // VB


def _pk_body(w_ref, e_ref, p_ref):
    p_ref[...] = jnp.sum(e_ref[...] * w_ref[...], axis=1)


_mesh = plsc.VectorSubcoreMesh(core_axis_name="c", subcore_axis_name="s")


@functools.partial(
    pl.kernel,
    mesh=_mesh,
    compiler_params=pltpu.CompilerParams(needs_layout_passes=False),
    out_type=jax.ShapeDtypeStruct((BATCH,), jnp.float32),
    scratch_types=[
        pltpu.VMEM((VOCAB,), jnp.float32),
        pltpu.VMEM((40, 128), jnp.int32),
        pltpu.VMEM((40, 128), jnp.int32),
        pltpu.VMEM((40, 128), jnp.int32),
        pltpu.VMEM((40, 128), jnp.float32),
        pltpu.VMEM((40, 128), jnp.float32),
        pltpu.VMEM((40, 128), jnp.float32),
        pltpu.VMEM((16,), jnp.float32),
        pltpu.VMEM((ROWS_PER_W,), jnp.float32),
        pltpu.SemaphoreType.DMA,
        pltpu.SemaphoreType.DMA,
        pltpu.SemaphoreType.DMA,
        pltpu.SemaphoreType.DMA,
        pltpu.SemaphoreType.DMA,
    ],
)
def _sc_pool(p_hbm, idsT_hbm, attT_hbm, b_hbm, out_hbm,
             p_v, ids_v0, ids_v1, ids_v2, att_v0, att_v1, att_v2, b_v, out_v,
             sem_p, sem_b, sem_g0, sem_g1, sem_g2):
    wid = lax.axis_index("s") * NC + lax.axis_index("c")
    base = wid * ROWS_PER_W
    bufs = [(ids_v0, att_v0, sem_g0), (ids_v1, att_v1, sem_g1),
            (ids_v2, att_v2, sem_g2)]
    CH = 40
    n_st = SEQ // CH

    def issue(st):
        iv, av, sg = bufs[st % 3]
        ci = pltpu.async_copy(
            idsT_hbm.at[pl.ds(st * CH, CH), pl.ds(base, ROWS_PER_W)], iv, sg)
        ca = pltpu.async_copy(
            attT_hbm.at[pl.ds(st * CH, CH), pl.ds(base, ROWS_PER_W)], av, sg)
        return ci, ca

    cp_p = pltpu.async_copy(p_hbm, p_v, sem_p)
    cp_b = pltpu.async_copy(b_hbm, b_v, sem_b)
    pend0 = issue(0)
    pend1 = issue(1)
    pends = [pend0, pend1, None]
    with jax.named_scope("p_wait"):
        cp_p.wait()
        cp_b.wait()
    bvec = b_v[...]
    zero = jnp.zeros((16,), jnp.float32)
    accs = [zero] * 8
    asums = [zero] * 8
    for st in range(n_st):
        iv_ref, av_ref, _ = bufs[st % 3]
        with jax.named_scope("g_wait"):
            pends[st % 3][0].wait()
            pends[st % 3][1].wait()
        if st + 2 < n_st:
            pends[(st + 2) % 3] = issue(st + 2)

        def body(s8, carry, _iv=iv_ref, _av=av_ref):
            acc, asum = list(carry[:8]), list(carry[8:])
            for lg in range(8):
                idv = _iv[s8, pl.ds(lg * 16, 16)]
                av = _av[s8, pl.ds(lg * 16, 16)]
                pv = plsc.load_gather(p_v, [idv])
                acc[lg] = acc[lg] + av * pv
                asum[lg] = asum[lg] + av
            return tuple(acc) + tuple(asum)

        with jax.named_scope("pool"):
            carry = lax.fori_loop(0, CH, body, tuple(accs) + tuple(asums))
            accs, asums = list(carry[:8]), list(carry[8:])
    for lg in range(8):
        x = accs[lg] / asums[lg] + bvec
        out_v[pl.ds(lg * 16, 16)] = 1.0 / (1.0 + jnp.exp(-x))
    pltpu.sync_copy(out_v, out_hbm.at[pl.ds(base, ROWS_PER_W)])


def kernel(ids, att_ids, embd, W, b):
    p = pl.pallas_call(
        _pk_body,
        grid=(TC_GRID,),
        in_specs=[
            pl.BlockSpec((1, DIM), lambda i: (0, 0)),
            pl.BlockSpec((VB, DIM), lambda i: (i, 0)),
        ],
        out_specs=pl.BlockSpec((VB,), lambda i: (i,)),
        out_shape=jax.ShapeDtypeStruct((VOCAB,), jnp.float32),
    )(W, embd)
    b16 = jnp.broadcast_to(b.astype(jnp.float32), (16,))
    return _sc_pool(p, ids.T, att_ids.T, b16)


# speedup vs baseline: 44.0199x; 1.3344x over previous
import functools

import jax
import jax.numpy as jnp
from jax import lax
from jax.experimental import pallas as pl
from jax.experimental.pallas import tpu as pltpu
from jax.experimental.pallas import tpu_sc as plsc

VOCAB = 100000
DIM = 128
BATCH = 4096
SEQ = 200

NC = 2
NS = 16
NW = NC * NS
ROWS_PER_W = BATCH // NW
GROUPS = ROWS_PER_W // 16

VB = 2048
VT = 48800
VSC = VOCAB - VT
ROWS_SC_W = VSC // NW
MV_CH = 160
MV_NCH = ROWS_SC_W // MV_CH
TC_GRID = (VT + VB - 1) // VB


def _pk_body(w_ref, e_ref, p_ref):
    p_ref[...] = jnp.sum(e_ref[...] * w_ref[...], axis=1)


_mesh = plsc.VectorSubcoreMesh(core_axis_name="c", subcore_axis_name="s")


@functools.partial(
    pl.kernel,
    mesh=_mesh,
    compiler_params=pltpu.CompilerParams(needs_layout_passes=False),
    out_type=jax.ShapeDtypeStruct((VSC,), jnp.float32),
    scratch_types=[
        pltpu.VMEM((MV_CH, DIM), jnp.float32),
        pltpu.VMEM((MV_CH, DIM), jnp.float32),
        pltpu.VMEM((MV_CH, DIM), jnp.float32),
        pltpu.VMEM((1, DIM), jnp.float32),
        pltpu.VMEM((ROWS_SC_W,), jnp.float32),
        pltpu.SemaphoreType.DMA,
        pltpu.SemaphoreType.DMA,
        pltpu.SemaphoreType.DMA,
        pltpu.SemaphoreType.DMA,
    ],
)
def _sc_matvec(embd_hbm, w_hbm, out_hbm,
               e_v0, e_v1, e_v2, w_v, out_v, sem_w, sem_e0, sem_e1, sem_e2):
    wid = lax.axis_index("s") * NC + lax.axis_index("c")
    rbase = VT + wid * ROWS_SC_W
    bufs = [(e_v0, sem_e0), (e_v1, sem_e1), (e_v2, sem_e2)]

    def issue(c):
        ev, se = bufs[c % 3]
        return pltpu.async_copy(
            embd_hbm.at[pl.ds(rbase + c * MV_CH, MV_CH), :], ev, se)

    cp_w = pltpu.async_copy(w_hbm, w_v, sem_w)
    pends = [issue(0), issue(1), None]
    cp_w.wait()
    wch = [w_v[0, pl.ds(k * 16, 16)] for k in range(8)]
    lane = lax.iota(jnp.int32, 16)
    zero = jnp.zeros((16,), jnp.float32)
    for c in range(MV_NCH):
        ev_ref, _ = bufs[c % 3]
        with jax.named_scope("mv_wait"):
            pends[c % 3].wait()
        if c + 2 < MV_NCH:
            pends[(c + 2) % 3] = issue(c + 2)

        def rg_body(rg, carry, _ev=ev_ref, _c=c):
            for rr in range(16):
                r = rg * 16 + rr
                acc = _ev[r, pl.ds(0, 16)] * wch[0]
                for k in range(1, 8):
                    acc = acc + _ev[r, pl.ds(k * 16, 16)] * wch[k]
                tot = jnp.sum(acc)
                if rr == 0:
                    out16 = jnp.full((16,), tot, dtype=jnp.float32)
                else:
                    out16 = jnp.where(lane == rr, tot, out16)
            out_v[pl.ds(_c * MV_CH + rg * 16, 16)] = out16
            return carry

        lax.fori_loop(0, MV_CH // 16, rg_body, zero)

    pltpu.sync_copy(out_v, out_hbm.at[pl.ds(wid * ROWS_SC_W, ROWS_SC_W)])


@functools.partial(
    pl.kernel,
    mesh=_mesh,
    compiler_params=pltpu.CompilerParams(needs_layout_passes=False),
    out_type=jax.ShapeDtypeStruct((BATCH,), jnp.float32),
    scratch_types=[
        pltpu.VMEM((VOCAB,), jnp.float32),
        pltpu.VMEM((40, 128), jnp.int32),
        pltpu.VMEM((40, 128), jnp.int32),
        pltpu.VMEM((40, 128), jnp.int32),
        pltpu.VMEM((40, 128), jnp.float32),
        pltpu.VMEM((40, 128), jnp.float32),
        pltpu.VMEM((40, 128), jnp.float32),
        pltpu.VMEM((16,), jnp.float32),
        pltpu.VMEM((ROWS_PER_W,), jnp.float32),
        pltpu.SemaphoreType.DMA,
        pltpu.SemaphoreType.DMA,
        pltpu.SemaphoreType.DMA,
        pltpu.SemaphoreType.DMA,
        pltpu.SemaphoreType.DMA,
    ],
)
def _sc_pool(ptc_hbm, psc_hbm, idsT_hbm, attT_hbm, b_hbm, out_hbm,
             p_v, ids_v0, ids_v1, ids_v2, att_v0, att_v1, att_v2, b_v, out_v,
             sem_p, sem_b, sem_g0, sem_g1, sem_g2):
    wid = lax.axis_index("s") * NC + lax.axis_index("c")
    base = wid * ROWS_PER_W
    bufs = [(ids_v0, att_v0, sem_g0), (ids_v1, att_v1, sem_g1),
            (ids_v2, att_v2, sem_g2)]
    CH = 40
    n_st = SEQ // CH

    def issue(st):
        iv, av, sg = bufs[st % 3]
        ci = pltpu.async_copy(
            idsT_hbm.at[pl.ds(st * CH, CH), pl.ds(base, ROWS_PER_W)], iv, sg)
        ca = pltpu.async_copy(
            attT_hbm.at[pl.ds(st * CH, CH), pl.ds(base, ROWS_PER_W)], av, sg)
        return ci, ca

    cp_p1 = pltpu.async_copy(ptc_hbm, p_v.at[pl.ds(0, VT)], sem_p)
    cp_p2 = pltpu.async_copy(psc_hbm, p_v.at[pl.ds(VT, VSC)], sem_p)
    cp_b = pltpu.async_copy(b_hbm, b_v, sem_b)
    pend0 = issue(0)
    pend1 = issue(1)
    pends = [pend0, pend1, None]
    with jax.named_scope("p_wait"):
        cp_p1.wait()
        cp_p2.wait()
        cp_b.wait()
    bvec = b_v[...]
    zero = jnp.zeros((16,), jnp.float32)
    accs = [zero] * 8
    asums = [zero] * 8
    for st in range(n_st):
        iv_ref, av_ref, _ = bufs[st % 3]
        with jax.named_scope("g_wait"):
            pends[st % 3][0].wait()
            pends[st % 3][1].wait()
        if st + 2 < n_st:
            pends[(st + 2) % 3] = issue(st + 2)

        def body(s8, carry, _iv=iv_ref, _av=av_ref):
            acc, asum = list(carry[:8]), list(carry[8:])
            for lg in range(8):
                idv = _iv[s8, pl.ds(lg * 16, 16)]
                av = _av[s8, pl.ds(lg * 16, 16)]
                pv = plsc.load_gather(p_v, [idv])
                acc[lg] = acc[lg] + av * pv
                asum[lg] = asum[lg] + av
            return tuple(acc) + tuple(asum)

        with jax.named_scope("pool"):
            carry = lax.fori_loop(0, CH, body, tuple(accs) + tuple(asums))
            accs, asums = list(carry[:8]), list(carry[8:])
    for lg in range(8):
        x = accs[lg] / asums[lg] + bvec
        out_v[pl.ds(lg * 16, 16)] = 1.0 / (1.0 + jnp.exp(-x))
    pltpu.sync_copy(out_v, out_hbm.at[pl.ds(base, ROWS_PER_W)])


def kernel(ids, att_ids, embd, W, b):
    p_sc = _sc_matvec(embd, W)
    p_tc = pl.pallas_call(
        _pk_body,
        grid=(TC_GRID,),
        in_specs=[
            pl.BlockSpec((1, DIM), lambda i: (0, 0)),
            pl.BlockSpec((VB, DIM), lambda i: (i, 0)),
        ],
        out_specs=pl.BlockSpec((VB,), lambda i: (i,)),
        out_shape=jax.ShapeDtypeStruct((VT,), jnp.float32),
    )(W, embd)
    b16 = jnp.broadcast_to(b.astype(jnp.float32), (16,))
    return _sc_pool(p_tc, p_sc, ids.T, att_ids.T, b16)


# speedup vs baseline: 44.9415x; 1.0209x over previous
import functools

import jax
import jax.numpy as jnp
from jax import lax
from jax.experimental import pallas as pl
from jax.experimental.pallas import tpu as pltpu
from jax.experimental.pallas import tpu_sc as plsc

VOCAB = 100000
DIM = 128
BATCH = 4096
SEQ = 200

NC = 2
NS = 16
NW = NC * NS
ROWS_PER_W = BATCH // NW
GROUPS = ROWS_PER_W // 16

VB = 2048
VT = 30880
VSC = VOCAB - VT
ROWS_SC_W = VSC // NW
MV_CH = 144
MV_NCH = ROWS_SC_W // MV_CH
TC_GRID = (VT + VB - 1) // VB


def _pk_body(w_ref, e_ref, p_ref):
    p_ref[...] = jnp.sum(e_ref[...] * w_ref[...], axis=1)


_mesh = plsc.VectorSubcoreMesh(core_axis_name="c", subcore_axis_name="s")


@functools.partial(
    pl.kernel,
    mesh=_mesh,
    compiler_params=pltpu.CompilerParams(needs_layout_passes=False),
    out_type=jax.ShapeDtypeStruct((VSC,), jnp.float32),
    scratch_types=[
        pltpu.VMEM((MV_CH, DIM), jnp.float32),
        pltpu.VMEM((MV_CH, DIM), jnp.float32),
        pltpu.VMEM((MV_CH, DIM), jnp.float32),
        pltpu.VMEM((1, DIM), jnp.float32),
        pltpu.VMEM((ROWS_SC_W,), jnp.float32),
        pltpu.SemaphoreType.DMA,
        pltpu.SemaphoreType.DMA,
        pltpu.SemaphoreType.DMA,
        pltpu.SemaphoreType.DMA,
    ],
)
def _sc_matvec(embd_hbm, w_hbm, out_hbm,
               e_v0, e_v1, e_v2, w_v, out_v, sem_w, sem_e0, sem_e1, sem_e2):
    wid = lax.axis_index("s") * NC + lax.axis_index("c")
    rbase = VT + wid * ROWS_SC_W
    bufs = [(e_v0, sem_e0), (e_v1, sem_e1), (e_v2, sem_e2)]

    def issue(c):
        ev, se = bufs[c % 3]
        return pltpu.async_copy(
            embd_hbm.at[pl.ds(rbase + c * MV_CH, MV_CH), :], ev, se)

    cp_w = pltpu.async_copy(w_hbm, w_v, sem_w)
    pends = [issue(0), issue(1), None]
    cp_w.wait()
    wch = [w_v[0, pl.ds(k * 16, 16)] for k in range(8)]
    lane = lax.iota(jnp.int32, 16)
    zero = jnp.zeros((16,), jnp.float32)
    for c in range(MV_NCH):
        ev_ref, _ = bufs[c % 3]
        with jax.named_scope("mv_wait"):
            pends[c % 3].wait()
        if c + 2 < MV_NCH:
            pends[(c + 2) % 3] = issue(c + 2)

        def rg_body(rg, carry, _ev=ev_ref, _c=c):
            for rr in range(16):
                r = rg * 16 + rr
                acc = _ev[r, pl.ds(0, 16)] * wch[0]
                for k in range(1, 8):
                    acc = acc + _ev[r, pl.ds(k * 16, 16)] * wch[k]
                tot = jnp.sum(acc)
                if rr == 0:
                    out16 = jnp.full((16,), tot, dtype=jnp.float32)
                else:
                    out16 = jnp.where(lane == rr, tot, out16)
            out_v[pl.ds(_c * MV_CH + rg * 16, 16)] = out16
            return carry

        lax.fori_loop(0, MV_CH // 16, rg_body, zero)

    pltpu.sync_copy(out_v, out_hbm.at[pl.ds(wid * ROWS_SC_W, ROWS_SC_W)])


@functools.partial(
    pl.kernel,
    mesh=_mesh,
    compiler_params=pltpu.CompilerParams(needs_layout_passes=False),
    out_type=jax.ShapeDtypeStruct((BATCH,), jnp.float32),
    scratch_types=[
        pltpu.VMEM((VOCAB,), jnp.float32),
        pltpu.VMEM((40, 128), jnp.int32),
        pltpu.VMEM((40, 128), jnp.int32),
        pltpu.VMEM((40, 128), jnp.int32),
        pltpu.VMEM((40, 128), jnp.float32),
        pltpu.VMEM((40, 128), jnp.float32),
        pltpu.VMEM((40, 128), jnp.float32),
        pltpu.VMEM((16,), jnp.float32),
        pltpu.VMEM((ROWS_PER_W,), jnp.float32),
        pltpu.SemaphoreType.DMA,
        pltpu.SemaphoreType.DMA,
        pltpu.SemaphoreType.DMA,
        pltpu.SemaphoreType.DMA,
        pltpu.SemaphoreType.DMA,
    ],
)
def _sc_pool(ptc_hbm, psc_hbm, idsT_hbm, attT_hbm, b_hbm, out_hbm,
             p_v, ids_v0, ids_v1, ids_v2, att_v0, att_v1, att_v2, b_v, out_v,
             sem_p, sem_b, sem_g0, sem_g1, sem_g2):
    wid = lax.axis_index("s") * NC + lax.axis_index("c")
    base = wid * ROWS_PER_W
    bufs = [(ids_v0, att_v0, sem_g0), (ids_v1, att_v1, sem_g1),
            (ids_v2, att_v2, sem_g2)]
    CH = 40
    n_st = SEQ // CH

    def issue(st):
        iv, av, sg = bufs[st % 3]
        ci = pltpu.async_copy(
            idsT_hbm.at[pl.ds(st * CH, CH), pl.ds(base, ROWS_PER_W)], iv, sg)
        ca = pltpu.async_copy(
            attT_hbm.at[pl.ds(st * CH, CH), pl.ds(base, ROWS_PER_W)], av, sg)
        return ci, ca

    cp_p1 = pltpu.async_copy(ptc_hbm, p_v.at[pl.ds(0, VT)], sem_p)
    cp_p2 = pltpu.async_copy(psc_hbm, p_v.at[pl.ds(VT, VSC)], sem_p)
    cp_b = pltpu.async_copy(b_hbm, b_v, sem_b)
    pend0 = issue(0)
    pend1 = issue(1)
    pends = [pend0, pend1, None]
    with jax.named_scope("p_wait"):
        cp_p1.wait()
        cp_p2.wait()
        cp_b.wait()
    bvec = b_v[...]
    zero = jnp.zeros((16,), jnp.float32)
    accs = [zero] * 8
    asums = [zero] * 8
    for st in range(n_st):
        iv_ref, av_ref, _ = bufs[st % 3]
        with jax.named_scope("g_wait"):
            pends[st % 3][0].wait()
            pends[st % 3][1].wait()
        if st + 2 < n_st:
            pends[(st + 2) % 3] = issue(st + 2)

        def body(s8, carry, _iv=iv_ref, _av=av_ref):
            acc, asum = list(carry[:8]), list(carry[8:])
            for lg in range(8):
                idv = _iv[s8, pl.ds(lg * 16, 16)]
                av = _av[s8, pl.ds(lg * 16, 16)]
                pv = plsc.load_gather(p_v, [idv])
                acc[lg] = acc[lg] + av * pv
                asum[lg] = asum[lg] + av
            return tuple(acc) + tuple(asum)

        with jax.named_scope("pool"):
            carry = lax.fori_loop(0, CH, body, tuple(accs) + tuple(asums))
            accs, asums = list(carry[:8]), list(carry[8:])
    for lg in range(8):
        x = accs[lg] / asums[lg] + bvec
        out_v[pl.ds(lg * 16, 16)] = 1.0 / (1.0 + jnp.exp(-x))
    pltpu.sync_copy(out_v, out_hbm.at[pl.ds(base, ROWS_PER_W)])


def kernel(ids, att_ids, embd, W, b):
    p_sc = _sc_matvec(embd, W)
    p_tc = pl.pallas_call(
        _pk_body,
        grid=(TC_GRID,),
        in_specs=[
            pl.BlockSpec((1, DIM), lambda i: (0, 0)),
            pl.BlockSpec((VB, DIM), lambda i: (i, 0)),
        ],
        out_specs=pl.BlockSpec((VB,), lambda i: (i,)),
        out_shape=jax.ShapeDtypeStruct((VT,), jnp.float32),
    )(W, embd)
    b16 = jnp.broadcast_to(b.astype(jnp.float32), (16,))
    return _sc_pool(p_tc, p_sc, ids.T, att_ids.T, b16)


# speedup vs baseline: 49.2605x; 1.0961x over previous
import functools

import jax
import jax.numpy as jnp
from jax import lax
from jax.experimental import pallas as pl
from jax.experimental.pallas import tpu as pltpu
from jax.experimental.pallas import tpu_sc as plsc

VOCAB = 100000
DIM = 128
BATCH = 4096
SEQ = 200

NC = 2
NS = 16
NW = NC * NS
ROWS_PER_W = BATCH // NW
GROUPS = ROWS_PER_W // 16

VB = 2048
VT = 30880
VSC = VOCAB - VT
ROWS_SC_W = VSC // NW
MV_CH = 144
MV_NCH = ROWS_SC_W // MV_CH
TC_GRID = (VT + VB - 1) // VB


def _pk_body(w_ref, e_ref, p_ref):
    p_ref[...] = jnp.sum(e_ref[...] * w_ref[...], axis=1)


_mesh = plsc.VectorSubcoreMesh(core_axis_name="c", subcore_axis_name="s")


@functools.partial(
    pl.kernel,
    mesh=_mesh,
    compiler_params=pltpu.CompilerParams(needs_layout_passes=False),
    out_type=jax.ShapeDtypeStruct((VSC,), jnp.float32),
    scratch_types=[
        pltpu.VMEM((MV_CH, DIM), jnp.float32),
        pltpu.VMEM((MV_CH, DIM), jnp.float32),
        pltpu.VMEM((MV_CH, DIM), jnp.float32),
        pltpu.VMEM((1, DIM), jnp.float32),
        pltpu.VMEM((ROWS_SC_W,), jnp.float32),
        pltpu.SemaphoreType.DMA,
        pltpu.SemaphoreType.DMA,
        pltpu.SemaphoreType.DMA,
        pltpu.SemaphoreType.DMA,
    ],
)
def _sc_matvec(embd_hbm, w_hbm, out_hbm,
               e_v0, e_v1, e_v2, w_v, out_v, sem_w, sem_e0, sem_e1, sem_e2):
    wid = lax.axis_index("s") * NC + lax.axis_index("c")
    rbase = VT + wid * ROWS_SC_W
    bufs = [(e_v0, sem_e0), (e_v1, sem_e1), (e_v2, sem_e2)]

    def issue(c):
        ev, se = bufs[c % 3]
        return pltpu.async_copy(
            embd_hbm.at[pl.ds(rbase + c * MV_CH, MV_CH), :], ev, se)

    cp_w = pltpu.async_copy(w_hbm, w_v, sem_w)
    pends = [issue(0), issue(1), None]
    cp_w.wait()
    wch = [w_v[0, pl.ds(k * 16, 16)] for k in range(8)]
    lane = lax.iota(jnp.int32, 16)
    zero = jnp.zeros((16,), jnp.float32)
    for c in range(MV_NCH):
        ev_ref, _ = bufs[c % 3]
        with jax.named_scope("mv_wait"):
            pends[c % 3].wait()
        if c + 2 < MV_NCH:
            pends[(c + 2) % 3] = issue(c + 2)

        def rg_body(rg, carry, _ev=ev_ref, _c=c):
            def row_body(rr, out16):
                r = rg * 16 + rr
                acc = _ev[r, pl.ds(0, 16)] * wch[0]
                for k in range(1, 8):
                    acc = acc + _ev[r, pl.ds(k * 16, 16)] * wch[k]
                tot = jnp.sum(acc)
                return jnp.where(lane == rr, tot, out16)

            out16 = plsc.parallel_loop(0, 16, unroll=4, carry=zero)(row_body)
            out_v[pl.ds(_c * MV_CH + rg * 16, 16)] = out16
            return carry

        lax.fori_loop(0, MV_CH // 16, rg_body, zero)

    pltpu.sync_copy(out_v, out_hbm.at[pl.ds(wid * ROWS_SC_W, ROWS_SC_W)])


@functools.partial(
    pl.kernel,
    mesh=_mesh,
    compiler_params=pltpu.CompilerParams(needs_layout_passes=False),
    out_type=jax.ShapeDtypeStruct((BATCH,), jnp.float32),
    scratch_types=[
        pltpu.VMEM((VOCAB,), jnp.float32),
        pltpu.VMEM((40, 128), jnp.int32),
        pltpu.VMEM((40, 128), jnp.int32),
        pltpu.VMEM((40, 128), jnp.int32),
        pltpu.VMEM((40, 128), jnp.float32),
        pltpu.VMEM((40, 128), jnp.float32),
        pltpu.VMEM((40, 128), jnp.float32),
        pltpu.VMEM((16,), jnp.float32),
        pltpu.VMEM((ROWS_PER_W,), jnp.float32),
        pltpu.SemaphoreType.DMA,
        pltpu.SemaphoreType.DMA,
        pltpu.SemaphoreType.DMA,
        pltpu.SemaphoreType.DMA,
        pltpu.SemaphoreType.DMA,
    ],
)
def _sc_pool(ptc_hbm, psc_hbm, idsT_hbm, attT_hbm, b_hbm, out_hbm,
             p_v, ids_v0, ids_v1, ids_v2, att_v0, att_v1, att_v2, b_v, out_v,
             sem_p, sem_b, sem_g0, sem_g1, sem_g2):
    wid = lax.axis_index("s") * NC + lax.axis_index("c")
    base = wid * ROWS_PER_W
    bufs = [(ids_v0, att_v0, sem_g0), (ids_v1, att_v1, sem_g1),
            (ids_v2, att_v2, sem_g2)]
    CH = 40
    n_st = SEQ // CH

    def issue(st):
        iv, av, sg = bufs[st % 3]
        ci = pltpu.async_copy(
            idsT_hbm.at[pl.ds(st * CH, CH), pl.ds(base, ROWS_PER_W)], iv, sg)
        ca = pltpu.async_copy(
            attT_hbm.at[pl.ds(st * CH, CH), pl.ds(base, ROWS_PER_W)], av, sg)
        return ci, ca

    cp_p1 = pltpu.async_copy(ptc_hbm, p_v.at[pl.ds(0, VT)], sem_p)
    cp_p2 = pltpu.async_copy(psc_hbm, p_v.at[pl.ds(VT, VSC)], sem_p)
    cp_b = pltpu.async_copy(b_hbm, b_v, sem_b)
    pend0 = issue(0)
    pend1 = issue(1)
    pends = [pend0, pend1, None]
    with jax.named_scope("p_wait"):
        cp_p1.wait()
        cp_p2.wait()
        cp_b.wait()
    bvec = b_v[...]
    zero = jnp.zeros((16,), jnp.float32)
    accs = [zero] * 8
    asums = [zero] * 8
    for st in range(n_st):
        iv_ref, av_ref, _ = bufs[st % 3]
        with jax.named_scope("g_wait"):
            pends[st % 3][0].wait()
            pends[st % 3][1].wait()
        if st + 2 < n_st:
            pends[(st + 2) % 3] = issue(st + 2)

        def body(s8, carry, _iv=iv_ref, _av=av_ref):
            acc, asum = list(carry[:8]), list(carry[8:])
            for lg in range(8):
                idv = _iv[s8, pl.ds(lg * 16, 16)]
                av = _av[s8, pl.ds(lg * 16, 16)]
                pv = plsc.load_gather(p_v, [idv])
                acc[lg] = acc[lg] + av * pv
                asum[lg] = asum[lg] + av
            return tuple(acc) + tuple(asum)

        with jax.named_scope("pool"):
            carry = lax.fori_loop(0, CH, body, tuple(accs) + tuple(asums))
            accs, asums = list(carry[:8]), list(carry[8:])
    for lg in range(8):
        x = accs[lg] / asums[lg] + bvec
        out_v[pl.ds(lg * 16, 16)] = 1.0 / (1.0 + jnp.exp(-x))
    pltpu.sync_copy(out_v, out_hbm.at[pl.ds(base, ROWS_PER_W)])


def kernel(ids, att_ids, embd, W, b):
    p_sc = _sc_matvec(embd, W)
    p_tc = pl.pallas_call(
        _pk_body,
        grid=(TC_GRID,),
        in_specs=[
            pl.BlockSpec((1, DIM), lambda i: (0, 0)),
            pl.BlockSpec((VB, DIM), lambda i: (i, 0)),
        ],
        out_specs=pl.BlockSpec((VB,), lambda i: (i,)),
        out_shape=jax.ShapeDtypeStruct((VT,), jnp.float32),
    )(W, embd)
    b16 = jnp.broadcast_to(b.astype(jnp.float32), (16,))
    return _sc_pool(p_tc, p_sc, ids.T, att_ids.T, b16)


# speedup vs baseline: 49.5740x; 1.0064x over previous
import functools

import jax
import jax.numpy as jnp
from jax import lax
from jax.experimental import pallas as pl
from jax.experimental.pallas import tpu as pltpu
from jax.experimental.pallas import tpu_sc as plsc

VOCAB = 100000
DIM = 128
BATCH = 4096
SEQ = 200

NC = 2
NS = 16
NW = NC * NS
ROWS_PER_W = BATCH // NW
GROUPS = ROWS_PER_W // 16

VB = 2048
VT = 30880
VSC = VOCAB - VT
ROWS_SC_W = VSC // NW
MV_CH = 144
MV_NCH = ROWS_SC_W // MV_CH
TC_GRID = (VT + VB - 1) // VB


def _pk_body(w_ref, e_ref, p_ref):
    p_ref[...] = jnp.sum(e_ref[...] * w_ref[...], axis=1)


_mesh = plsc.VectorSubcoreMesh(core_axis_name="c", subcore_axis_name="s")


@functools.partial(
    pl.kernel,
    mesh=_mesh,
    compiler_params=pltpu.CompilerParams(needs_layout_passes=False),
    out_type=jax.ShapeDtypeStruct((VSC,), jnp.float32),
    scratch_types=[
        pltpu.VMEM((MV_CH, DIM), jnp.float32),
        pltpu.VMEM((MV_CH, DIM), jnp.float32),
        pltpu.VMEM((MV_CH, DIM), jnp.float32),
        pltpu.VMEM((1, DIM), jnp.float32),
        pltpu.VMEM((ROWS_SC_W,), jnp.float32),
        pltpu.SemaphoreType.DMA,
        pltpu.SemaphoreType.DMA,
        pltpu.SemaphoreType.DMA,
        pltpu.SemaphoreType.DMA,
    ],
)
def _sc_matvec(embd_hbm, w_hbm, out_hbm,
               e_v0, e_v1, e_v2, w_v, out_v, sem_w, sem_e0, sem_e1, sem_e2):
    wid = lax.axis_index("s") * NC + lax.axis_index("c")
    rbase = VT + wid * ROWS_SC_W
    bufs = [(e_v0, sem_e0), (e_v1, sem_e1), (e_v2, sem_e2)]

    def issue(c):
        ev, se = bufs[c % 3]
        return pltpu.async_copy(
            embd_hbm.at[pl.ds(rbase + c * MV_CH, MV_CH), :], ev, se)

    cp_w = pltpu.async_copy(w_hbm, w_v, sem_w)
    pends = [issue(0), issue(1), None]
    cp_w.wait()
    wch = [w_v[0, pl.ds(k * 16, 16)] for k in range(8)]
    lane = lax.iota(jnp.int32, 16)
    zero = jnp.zeros((16,), jnp.float32)
    for c in range(MV_NCH):
        ev_ref, _ = bufs[c % 3]
        with jax.named_scope("mv_wait"):
            pends[c % 3].wait()
        if c + 2 < MV_NCH:
            pends[(c + 2) % 3] = issue(c + 2)

        def rg_body(rg, carry, _ev=ev_ref, _c=c):
            def row_body(rr, out16):
                r = rg * 16 + rr
                acc = _ev[r, pl.ds(0, 16)] * wch[0]
                for k in range(1, 8):
                    acc = acc + _ev[r, pl.ds(k * 16, 16)] * wch[k]
                tot = jnp.sum(acc)
                return jnp.where(lane == rr, tot, out16)

            out16 = plsc.parallel_loop(0, 16, unroll=4, carry=zero)(row_body)
            out_v[pl.ds(_c * MV_CH + rg * 16, 16)] = out16
            return carry

        lax.fori_loop(0, MV_CH // 16, rg_body, zero)

    pltpu.sync_copy(out_v, out_hbm.at[pl.ds(wid * ROWS_SC_W, ROWS_SC_W)])


@functools.partial(
    pl.kernel,
    mesh=_mesh,
    compiler_params=pltpu.CompilerParams(needs_layout_passes=False),
    out_type=jax.ShapeDtypeStruct((BATCH,), jnp.float32),
    scratch_types=[
        pltpu.VMEM((VOCAB,), jnp.float32),
        pltpu.VMEM((40, 128), jnp.int32),
        pltpu.VMEM((40, 128), jnp.int32),
        pltpu.VMEM((40, 128), jnp.int32),
        pltpu.VMEM((40, 128), jnp.float32),
        pltpu.VMEM((40, 128), jnp.float32),
        pltpu.VMEM((40, 128), jnp.float32),
        pltpu.VMEM((16,), jnp.float32),
        pltpu.VMEM((ROWS_PER_W,), jnp.float32),
        pltpu.SemaphoreType.DMA,
        pltpu.SemaphoreType.DMA,
        pltpu.SemaphoreType.DMA,
        pltpu.SemaphoreType.DMA,
        pltpu.SemaphoreType.DMA,
    ],
)
def _sc_pool(ptc_hbm, psc_hbm, idsT_hbm, attT_hbm, b_hbm, out_hbm,
             p_v, ids_v0, ids_v1, ids_v2, att_v0, att_v1, att_v2, b_v, out_v,
             sem_p, sem_b, sem_g0, sem_g1, sem_g2):
    wid = lax.axis_index("s") * NC + lax.axis_index("c")
    base = wid * ROWS_PER_W
    bufs = [(ids_v0, att_v0, sem_g0), (ids_v1, att_v1, sem_g1),
            (ids_v2, att_v2, sem_g2)]
    CH = 40
    n_st = SEQ // CH

    def issue(st):
        iv, av, sg = bufs[st % 3]
        ci = pltpu.async_copy(
            idsT_hbm.at[pl.ds(st * CH, CH), pl.ds(base, ROWS_PER_W)], iv, sg)
        ca = pltpu.async_copy(
            attT_hbm.at[pl.ds(st * CH, CH), pl.ds(base, ROWS_PER_W)], av, sg)
        return ci, ca

    p_cps = []
    for j in range(10):
        pc = lax.rem(wid + j, 10) * 3088
        p_cps.append(pltpu.async_copy(
            ptc_hbm.at[pl.ds(pc, 3088)], p_v.at[pl.ds(pc, 3088)], sem_p))
    for j in range(16):
        pc = lax.rem(wid + j, 16) * 4320
        p_cps.append(pltpu.async_copy(
            psc_hbm.at[pl.ds(pc, 4320)], p_v.at[pl.ds(VT + pc, 4320)], sem_p))
    cp_b = pltpu.async_copy(b_hbm, b_v, sem_b)
    pend0 = issue(0)
    pend1 = issue(1)
    pends = [pend0, pend1, None]
    with jax.named_scope("p_wait"):
        for cp in p_cps:
            cp.wait()
        cp_b.wait()
    bvec = b_v[...]
    zero = jnp.zeros((16,), jnp.float32)
    accs = [zero] * 8
    asums = [zero] * 8
    for st in range(n_st):
        iv_ref, av_ref, _ = bufs[st % 3]
        with jax.named_scope("g_wait"):
            pends[st % 3][0].wait()
            pends[st % 3][1].wait()
        if st + 2 < n_st:
            pends[(st + 2) % 3] = issue(st + 2)

        def body(s8, carry, _iv=iv_ref, _av=av_ref):
            acc, asum = list(carry[:8]), list(carry[8:])
            for lg in range(8):
                idv = _iv[s8, pl.ds(lg * 16, 16)]
                av = _av[s8, pl.ds(lg * 16, 16)]
                pv = plsc.load_gather(p_v, [idv])
                acc[lg] = acc[lg] + av * pv
                asum[lg] = asum[lg] + av
            return tuple(acc) + tuple(asum)

        with jax.named_scope("pool"):
            carry = lax.fori_loop(0, CH, body, tuple(accs) + tuple(asums))
            accs, asums = list(carry[:8]), list(carry[8:])
    for lg in range(8):
        x = accs[lg] / asums[lg] + bvec
        out_v[pl.ds(lg * 16, 16)] = 1.0 / (1.0 + jnp.exp(-x))
    pltpu.sync_copy(out_v, out_hbm.at[pl.ds(base, ROWS_PER_W)])


def kernel(ids, att_ids, embd, W, b):
    p_sc = _sc_matvec(embd, W)
    p_tc = pl.pallas_call(
        _pk_body,
        grid=(TC_GRID,),
        in_specs=[
            pl.BlockSpec((1, DIM), lambda i: (0, 0)),
            pl.BlockSpec((VB, DIM), lambda i: (i, 0)),
        ],
        out_specs=pl.BlockSpec((VB,), lambda i: (i,)),
        out_shape=jax.ShapeDtypeStruct((VT,), jnp.float32),
    )(W, embd)
    b16 = jnp.broadcast_to(b.astype(jnp.float32), (16,))
    return _sc_pool(p_tc, p_sc, ids.T, att_ids.T, b16)
